# Initial kernel scaffold; baseline (speedup 1.0000x reference)
import jax
import jax.numpy as jnp
from jax.experimental import pallas as pl


def kernel(batsman_idx, bowler_idx, numeric, bat_table, bwl_table, W0, b0, g0, be0, W1, b1, g1, be1, W2, b2, g2, be2, W3, b3, g3, be3, W4, b4):
    raise NotImplementedError("write your pallas kernel here")


# speedup vs baseline: 1.1384x; 1.1384x over previous
import functools

import jax
import jax.numpy as jnp
from jax import lax
from jax.experimental import pallas as pl
from jax.experimental.pallas import tpu as pltpu
from jax.experimental.pallas import tpu_sc as plsc

B = 16384
D = 16
NW = 32
BPW = B // NW
CH = 128
NCH = BPW // CH
EPS = 1e-5


def _gather_body(bat_idx, bwl_idx, bat_tab, bwl_tab, bat_out, bwl_out,
                 idx_v, rows_v, sem):
    c = lax.axis_index("c")
    s = lax.axis_index("s")
    wid = s * 2 + c
    base = wid * BPW
    irow = wid * NCH
    pltpu.sync_copy(bat_idx.at[pl.ds(irow, NCH)], idx_v.at[0])
    pltpu.sync_copy(bwl_idx.at[pl.ds(irow, NCH)], idx_v.at[1])
    copies = []
    for t, tab in ((0, bat_tab), (1, bwl_tab)):
        for j in range(NCH):
            copies.append(
                pltpu.async_copy(tab.at[idx_v.at[t, j]],
                                 rows_v.at[t, pl.ds(j * CH, CH)], sem))
    for cp in copies:
        cp.wait()
    pltpu.sync_copy(rows_v.at[0], bat_out.at[pl.ds(base, BPW)])
    pltpu.sync_copy(rows_v.at[1], bwl_out.at[pl.ds(base, BPW)])


@functools.cache
def _make_gather():
    return pl.kernel(
        _gather_body,
        out_type=(jax.ShapeDtypeStruct((B, D), jnp.float32),
                  jax.ShapeDtypeStruct((B, D), jnp.float32)),
        mesh=plsc.VectorSubcoreMesh(core_axis_name="c", subcore_axis_name="s"),
        scratch_types=[
            pltpu.VMEM((2, NCH, CH), jnp.int32),
            pltpu.VMEM((2, BPW, D), jnp.float32),
            pltpu.SemaphoreType.DMA,
        ],
        compiler_params=pltpu.CompilerParams(use_tc_tiling_on_sc=False),
    )


def _bn_relu(z, g, be):
    m = jnp.mean(z, axis=0, keepdims=True)
    zc = z - m
    v = jnp.mean(zc * zc, axis=0, keepdims=True)
    return jnp.maximum(zc * lax.rsqrt(v + EPS) * g + be, 0.0)


def _mlp_body(bat, bwl, num, w0a, w0b, w0c, b0, g0, be0,
              w1, b1, g1, be1, w2, b2, g2, be2, w3, b3, g3, be3,
              w4, b4, out):
    f32 = jnp.float32
    z = (jnp.dot(bat[...], w0a[...], preferred_element_type=f32)
         + jnp.dot(bwl[...], w0b[...], preferred_element_type=f32)
         + jnp.dot(num[...], w0c[...], preferred_element_type=f32)
         + b0[...])
    h = _bn_relu(z, g0[...], be0[...])
    z = jnp.dot(h, w1[...], preferred_element_type=f32) + b1[...]
    h = _bn_relu(z, g1[...], be1[...])
    z = jnp.dot(h, w2[...], preferred_element_type=f32) + b2[...]
    h = _bn_relu(z, g2[...], be2[...])
    z = jnp.dot(h, w3[...], preferred_element_type=f32) + b3[...]
    h = _bn_relu(z, g3[...], be3[...])
    out[...] = jnp.dot(h, w4[...], preferred_element_type=f32) + b4[...]


_mlp = pl.pallas_call(
    _mlp_body,
    out_shape=jax.ShapeDtypeStruct((B, 1), jnp.float32),
    compiler_params=pltpu.CompilerParams(vmem_limit_bytes=128 * 1024 * 1024),
)


def kernel(batsman_idx, bowler_idx, numeric, bat_table, bwl_table,
           W0, b0, g0, be0, W1, b1, g1, be1, W2, b2, g2, be2,
           W3, b3, g3, be3, W4, b4):
    bat_idx = batsman_idx.astype(jnp.int32).reshape(NW * NCH, CH)
    bwl_idx = bowler_idx.astype(jnp.int32).reshape(NW * NCH, CH)
    bat_emb, bwl_emb = _make_gather()(bat_idx, bwl_idx, bat_table, bwl_table)
    args = (bat_emb, bwl_emb, numeric,
            W0[:, :D].T, W0[:, D:2 * D].T, W0[:, 2 * D:].T,
            b0.reshape(1, -1), g0.reshape(1, -1), be0.reshape(1, -1),
            W1.T, b1.reshape(1, -1), g1.reshape(1, -1), be1.reshape(1, -1),
            W2.T, b2.reshape(1, -1), g2.reshape(1, -1), be2.reshape(1, -1),
            W3.T, b3.reshape(1, -1), g3.reshape(1, -1), be3.reshape(1, -1),
            W4.T, b4.reshape(1, -1))
    return _mlp(*args).reshape(B)


# speedup vs baseline: 1.2928x; 1.1356x over previous
import functools

import jax
import jax.numpy as jnp
from jax import lax
from jax.experimental import pallas as pl
from jax.experimental.pallas import tpu as pltpu
from jax.experimental.pallas import tpu_sc as plsc

B = 16384
D = 16
NW = 32
BPW = B // NW
CH = 128
NCH = BPW // CH
PG = 8
PR = B // PG
EPS = 1e-5
HID = (64, 32, 16, 8)


def _gather_body(bat_idx, bwl_idx, bat_tab, bwl_tab, bat_out, bwl_out,
                 idx_v, rows_bat, rows_bwl, sem):
    c = lax.axis_index("c")
    s = lax.axis_index("s")
    wid = s * 2 + c
    irow = wid * NCH
    pltpu.sync_copy(bat_idx.at[pl.ds(irow, NCH)], idx_v.at[0])
    pltpu.sync_copy(bwl_idx.at[pl.ds(irow, NCH)], idx_v.at[1])
    copies = []
    for t, rows in ((0, rows_bat), (1, rows_bwl)):
        tab = bat_tab if t == 0 else bwl_tab
        for j in range(NCH):
            copies.append(
                pltpu.async_copy(tab.at[idx_v.at[t, j]],
                                 rows.at[pl.ds(j * CH, CH)], sem))
    for cp in copies:
        cp.wait()
    base = wid * BPW
    pltpu.sync_copy(rows_bat, bat_out.at[pl.ds(base, BPW)])
    pltpu.sync_copy(rows_bwl, bwl_out.at[pl.ds(base, BPW)])


@functools.cache
def _make_gather():
    return pl.kernel(
        _gather_body,
        out_type=(jax.ShapeDtypeStruct((B, D), jnp.float32),
                  jax.ShapeDtypeStruct((B, D), jnp.float32)),
        mesh=plsc.VectorSubcoreMesh(core_axis_name="c", subcore_axis_name="s"),
        scratch_types=[
            pltpu.VMEM((2, NCH, CH), jnp.int32),
            pltpu.VMEM((BPW, D), jnp.float32),
            pltpu.VMEM((BPW, D), jnp.float32),
            pltpu.SemaphoreType.DMA,
        ],
        compiler_params=pltpu.CompilerParams(use_tc_tiling_on_sc=False),
    )


def _fold_tile(s, d):
    m = s[:, 0:d]
    for g in range(1, PG):
        m = m + s[:, g * d:(g + 1) * d]
    return jnp.concatenate([m] * PG, axis=1)


def _bn_relu(z, d, gt, bet):
    ones = jnp.ones((1, PR), jnp.float32)
    s1 = jnp.dot(ones, z, preferred_element_type=jnp.float32)
    s2 = jnp.dot(ones, z * z, preferred_element_type=jnp.float32)
    m = _fold_tile(s1, d) * (1.0 / B)
    e2 = _fold_tile(s2, d) * (1.0 / B)
    var = e2 - m * m
    scale = gt * lax.rsqrt(var + EPS)
    shift = bet - m * scale
    return jnp.maximum(z * scale + shift, 0.0)


def _mlp_body(bat_p, bwl_p, num_p, k0a, k0b, k0c, g0, be0,
              k1, g1, be1, k2, g2, be2, k3, g3, be3, k4, b4, out):
    f32 = jnp.float32
    z = (jnp.dot(bat_p[...], k0a[...], preferred_element_type=f32)
         + jnp.dot(bwl_p[...], k0b[...], preferred_element_type=f32)
         + jnp.dot(num_p[...], k0c[...], preferred_element_type=f32))
    h = _bn_relu(z, HID[0], g0[...], be0[...])
    z = jnp.dot(h, k1[...], preferred_element_type=f32)
    h = _bn_relu(z, HID[1], g1[...], be1[...])
    z = jnp.dot(h, k2[...], preferred_element_type=f32)
    h = _bn_relu(z, HID[2], g2[...], be2[...])
    z = jnp.dot(h, k3[...], preferred_element_type=f32)
    h = _bn_relu(z, HID[3], g3[...], be3[...])
    out[...] = jnp.dot(h, k4[...], preferred_element_type=f32) + b4[...]


_mlp = pl.pallas_call(
    _mlp_body,
    out_shape=jax.ShapeDtypeStruct((PR, PG), jnp.float32),
    compiler_params=pltpu.CompilerParams(vmem_limit_bytes=100 * 1024 * 1024),
)


def _kron8(w):
    return jnp.kron(jnp.eye(PG, dtype=jnp.float32), w)


def _tile8(v):
    return jnp.tile(v, PG).reshape(1, -1)


def kernel(batsman_idx, bowler_idx, numeric, bat_table, bwl_table,
           W0, b0, g0, be0, W1, b1, g1, be1, W2, b2, g2, be2,
           W3, b3, g3, be3, W4, b4):
    bat_idx = batsman_idx.astype(jnp.int32).reshape(B // CH, CH)
    bwl_idx = bowler_idx.astype(jnp.int32).reshape(B // CH, CH)
    bat_emb, bwl_emb = _make_gather()(bat_idx, bwl_idx, bat_table, bwl_table)
    bat_p = bat_emb.reshape(PR, PG * D)
    bwl_p = bwl_emb.reshape(PR, PG * D)
    num_p = jnp.concatenate(
        [numeric, jnp.zeros((B, 1), jnp.float32)], axis=1).reshape(PR, PG * 8)
    w0c = jnp.pad(W0[:, 2 * D:].T, ((0, 1), (0, 0)))
    args = (bat_p, bwl_p, num_p,
            _kron8(W0[:, :D].T), _kron8(W0[:, D:2 * D].T), _kron8(w0c),
            _tile8(g0), _tile8(be0),
            _kron8(W1.T), _tile8(g1), _tile8(be1),
            _kron8(W2.T), _tile8(g2), _tile8(be2),
            _kron8(W3.T), _tile8(g3), _tile8(be3),
            _kron8(W4.T), b4.reshape(1, 1))
    return _mlp(*args).reshape(B)
